# baseline (speedup 1.0000x reference)
import functools

import jax
import jax.numpy as jnp
from jax import lax
from jax.experimental import pallas as pl
from jax.experimental.pallas import tpu as pltpu
from jax.experimental.pallas import tpu_sc as plsc

F32 = jnp.float32
BF16 = jnp.bfloat16


def _dot(a, b):
    return jnp.dot(a, b, preferred_element_type=F32)


def _stage_a_node_body(x_ref, wm_ref, wab_ref, ba_ref, xm_ref, xb_ref):
    x = x_ref[...]
    xm_ref[...] = _dot(x, wm_ref[...])
    xb_ref[...] = _dot(x, wab_ref[...]) + ba_ref[...]


def _pack_bf16_pair(lo_f32, hi_f32):
    lo = lax.bitcast_convert_type(lo_f32.astype(BF16), jnp.uint16)
    hi = lax.bitcast_convert_type(hi_f32.astype(BF16), jnp.uint16)
    word = lo.astype(jnp.uint32) | (hi.astype(jnp.uint32) << 16)
    return lax.bitcast_convert_type(word, jnp.int32)


def _stage_a_edge_body(ea_ref, w1_ref, b1_ref, w2_ref, b2_ref,
                       em1_ref, em2_ref):
    ea = ea_ref[...]

    def em(w_ref, b_ref):
        w = w_ref[...]
        b = b_ref[...]
        half = w.shape[1] // 2
        lo = _dot(ea, w[:, :half]) + b[:, :half]
        hi = _dot(ea, w[:, half:]) + b[:, half:]
        return _pack_bf16_pair(lo, hi)

    em1_ref[...] = em(w1_ref, b1_ref)
    em2_ref[...] = em(w2_ref, b2_ref)


def _stage_b_body(agg_ref, xb_ref, wat_ref, wm2_ref, wa2b_ref, ba2_ref,
                  xm2_ref, hb2_ref):
    aggr = agg_ref[0] + agg_ref[1]
    h = jnp.maximum(_dot(aggr, wat_ref[...]) + xb_ref[...], 0.0)
    xm2_ref[...] = _dot(h, wm2_ref[...])
    hb2_ref[...] = _dot(h, wa2b_ref[...]) + ba2_ref[...]


def _stage_c_body(agg_ref, hb_ref, wat_ref, out_ref):
    aggr = agg_ref[0] + agg_ref[1]
    out_ref[...] = jnp.maximum(_dot(aggr, wat_ref[...]) + hb_ref[...], 0.0)


def _node_block(bn, h):
    return pl.BlockSpec((bn, h), lambda i: (i, 0))


def _full_block(shape):
    nd = len(shape)
    return pl.BlockSpec(shape, lambda i: (0,) * nd)


def _stage_a_node(x, wm_top, wa_bot, ba, bn):
    n, d = x.shape
    h = wm_top.shape[1]
    grid = n // bn
    return pl.pallas_call(
        _stage_a_node_body,
        grid=(grid,),
        in_specs=[_node_block(bn, d), _full_block(wm_top.shape),
                  _full_block(wa_bot.shape), _full_block(ba.shape)],
        out_specs=[_node_block(bn, h), _node_block(bn, h)],
        out_shape=[jax.ShapeDtypeStruct((n, h), F32),
                   jax.ShapeDtypeStruct((n, h), F32)],
    )(x, wm_top, wa_bot, ba)


def _stage_a_edge(ea, w1, b1, w2, b2, be):
    e, ed = ea.shape
    h = w1.shape[1]
    grid = e // be
    return pl.pallas_call(
        _stage_a_edge_body,
        grid=(grid,),
        in_specs=[_node_block(be, ed), _full_block(w1.shape),
                  _full_block(b1.shape), _full_block(w2.shape),
                  _full_block(b2.shape)],
        out_specs=[_node_block(be, h // 2), _node_block(be, h // 2)],
        out_shape=[jax.ShapeDtypeStruct((e, h // 2), jnp.int32),
                   jax.ShapeDtypeStruct((e, h // 2), jnp.int32)],
    )(ea, w1, b1, w2, b2)


def _stage_b(agg, xb, wa_top, wm2_top, wa2_bot, ba2, bn):
    n, h = xb.shape
    o = wm2_top.shape[1]
    grid = n // bn
    agg_spec = pl.BlockSpec((2, bn, h), lambda i: (0, i, 0))
    return pl.pallas_call(
        _stage_b_body,
        grid=(grid,),
        in_specs=[agg_spec, _node_block(bn, h), _full_block(wa_top.shape),
                  _full_block(wm2_top.shape), _full_block(wa2_bot.shape),
                  _full_block(ba2.shape)],
        out_specs=[_node_block(bn, o), _node_block(bn, o)],
        out_shape=[jax.ShapeDtypeStruct((n, o), F32),
                   jax.ShapeDtypeStruct((n, o), F32)],
    )(agg, xb, wa_top, wm2_top, wa2_bot, ba2)


def _stage_c(agg, hb, wa_top, bn):
    n, o = hb.shape
    grid = n // bn
    agg_spec = pl.BlockSpec((2, bn, o), lambda i: (0, i, 0))
    return pl.pallas_call(
        _stage_c_body,
        grid=(grid,),
        in_specs=[agg_spec, _node_block(bn, o), _full_block(wa_top.shape)],
        out_specs=_node_block(bn, o),
        out_shape=jax.ShapeDtypeStruct((n, o), F32),
    )(agg, hb, wa_top)


def _sc_edge_pass(xm, em, src, dst, *, block_e):
    n, h = xm.shape
    e = src.shape[0]
    mesh = plsc.VectorSubcoreMesh(core_axis_name="c", subcore_axis_name="s")
    n_tiles = mesh.num_cores * mesh.num_subcores
    n_sub = mesh.num_subcores
    chunk = e // n_tiles
    nblk = chunk // block_e
    assert chunk * n_tiles == e and nblk * block_e == chunk
    rows_base = (n // n_sub) // 8 * 8
    tail_rows = n - rows_base * n_sub
    assert rows_base % 8 == 0 and tail_rows % 8 == 0
    nfeat = h // 16

    @functools.partial(
        pl.kernel,
        out_type=jax.ShapeDtypeStruct((mesh.num_cores, n, h), F32),
        mesh=mesh,
        scratch_types=[
            pltpu.VMEM_SHARED((n, h), F32),
            pltpu.VMEM((2, block_e), jnp.int32),
            pltpu.VMEM((block_e,), jnp.int32),
            pltpu.VMEM((block_e,), jnp.int32),
            pltpu.VMEM((block_e, h // 2), jnp.int32),
            pltpu.VMEM((block_e, h), F32),
            pltpu.VMEM((block_e, h), F32),
            pltpu.VMEM((block_e, h), F32),
            pltpu.SemaphoreType.DMA,
            pltpu.SemaphoreType.DMA,
            pltpu.SemaphoreType.DMA,
            pltpu.SemaphoreType.DMA,
            pltpu.SemaphoreType.DMA,
            pltpu.SemaphoreType.DMA,
            pltpu.SemaphoreType.DMA,
            pltpu.SemaphoreType.DMA,
        ],
    )
    def edge_kernel(xm_hbm, em_hbm, src_hbm, dst_hbm, out_hbm,
                    acc, dbuf, si0, si1, embuf, gb0, gb1, obuf,
                    ssi0, ssi1, sd0, sd1, sem_e, sg0, sg1, sem_s):
        sbuf = (si0, si1)
        gbuf = (gb0, gb1)
        sem_i = (ssi0, ssi1)
        sem_d = (sd0, sd1)
        sem_g = (sg0, sg1)
        c = lax.axis_index("c")
        s = lax.axis_index("s")
        wid = c * n_sub + s

        zero = jnp.zeros((16,), F32)

        def zero_row(k, carry):
            for j in range(nfeat):
                obuf[k, pl.ds(j * 16, 16)] = zero
            return carry

        lax.fori_loop(0, block_e, zero_row, 0)
        r0 = s * rows_base
        nfull = rows_base // block_e
        rem = rows_base - nfull * block_e
        for t in range(nfull):
            pltpu.sync_copy(obuf, acc.at[pl.ds(r0 + t * block_e, block_e)])
        if rem:
            pltpu.sync_copy(obuf.at[pl.ds(0, rem)],
                            acc.at[pl.ds(r0 + nfull * block_e, rem)])
        if tail_rows:
            @pl.when(s == n_sub - 1)
            def _zero_tail():
                pltpu.sync_copy(obuf.at[pl.ds(0, tail_rows)],
                                acc.at[pl.ds(rows_base * n_sub, tail_rows)])
        plsc.subcore_barrier()

        base = wid * chunk

        def em_slice(i):
            return em_hbm.at[pl.ds(base + i * block_e, block_e)]

        def src_slice(i):
            return src_hbm.at[pl.ds(base + i * block_e, block_e)]

        def dst_slice(i):
            return dst_hbm.at[pl.ds(base + i * block_e, block_e)]

        def compute(b):
            g, m = gbuf[b], embuf

            def row(k, carry):
                for j in range(h // 32):
                    vw = m[k, pl.ds(j * 16, 16)]
                    ma = lax.bitcast_convert_type(
                        lax.shift_left(vw, 16), F32)
                    mb = lax.bitcast_convert_type(
                        lax.shift_left(
                            lax.shift_right_logical(vw, 16), 16), F32)
                    sla = pl.ds(j * 32, 16)
                    slb = pl.ds(j * 32 + 16, 16)
                    obuf[k, sla] = jnp.maximum(g[k, sla] + ma, 0.0)
                    obuf[k, slb] = jnp.maximum(g[k, slb] + mb, 0.0)
                return carry

            lax.fori_loop(0, block_e, row, 0)

        pltpu.sync_copy(src_slice(0), sbuf[0])
        pltpu.sync_copy(dst_slice(0), dbuf.at[0])
        pltpu.async_copy(xm_hbm.at[sbuf[0]], gbuf[0], sem_g[0])
        pltpu.async_copy(em_slice(0), embuf, sem_e)
        if nblk > 1:
            pltpu.async_copy(src_slice(1), sbuf[1], sem_i[1])

        def outer(t, carry):
            for b in range(2):
                i = t * 2 + b
                nxt = 1 - b

                @pl.when(i < nblk)
                def _block():
                    _run_block(i, b, nxt)
            return carry

        def _run_block(i, b, nxt):
            if True:
                @pl.when(i + 1 < nblk)
                def _issue_next():
                    pltpu.make_async_copy(src_slice(i + 1), sbuf[nxt],
                                          sem_i[nxt]).wait()
                    pltpu.async_copy(xm_hbm.at[sbuf[nxt]],
                                     gbuf[nxt], sem_g[nxt])

                pltpu.make_async_copy(xm_hbm.at[sbuf[b]],
                                      gbuf[b], sem_g[b]).wait()
                pltpu.make_async_copy(em_slice(i), embuf, sem_e).wait()

                @pl.when(i + 2 < nblk)
                def _prefetch_src():
                    pltpu.async_copy(src_slice(i + 2), sbuf[b], sem_i[b])

                @pl.when(i > 0)
                def _wait_prev_scatter():
                    pltpu.make_async_copy(obuf, acc.at[dbuf.at[nxt]],
                                          sem_s).wait()

                @pl.when(i + 1 < nblk)
                def _prefetch_dst():
                    pltpu.async_copy(dst_slice(i + 1), dbuf.at[nxt],
                                     sem_d[nxt])

                compute(b)

                @pl.when(i > 0)
                def _wait_dst():
                    pltpu.make_async_copy(dst_slice(i), dbuf.at[b],
                                          sem_d[b]).wait()

                pltpu.async_copy(obuf, acc.at[dbuf.at[b]], sem_s, add=True)

                @pl.when(i + 1 < nblk)
                def _issue_next_em():
                    pltpu.async_copy(em_slice(i + 1), embuf, sem_e)

        lax.fori_loop(0, (nblk + 1) // 2, outer, 0)
        last = nblk - 1
        pltpu.make_async_copy(obuf, acc.at[dbuf.at[last % 2]], sem_s).wait()
        plsc.subcore_barrier()

        pltpu.sync_copy(acc.at[pl.ds(r0, rows_base)],
                        out_hbm.at[c, pl.ds(r0, rows_base)])
        if tail_rows:
            @pl.when(s == n_sub - 1)
            def _write_tail():
                pltpu.sync_copy(acc.at[pl.ds(rows_base * n_sub, tail_rows)],
                                out_hbm.at[c, pl.ds(rows_base * n_sub,
                                                    tail_rows)])

    return edge_kernel(xm, em, src, dst)


def kernel(x, edge_index, edge_attr, Wm1, bm1, Wa1, ba1, Wm2, bm2, Wa2, ba2):
    n, d = x.shape
    e, ed = edge_attr.shape
    h = Wm1.shape[1]
    o = Wm2.shape[1]
    src = edge_index[0]
    dst = edge_index[1]

    bn = 1000 if n % 1000 == 0 else 8
    be = 2000 if e % 2000 == 0 else 8
    block_e = 80

    cperm = jnp.asarray(
        [32 * j + t for j in range(h // 32) for t in range(16)]
        + [32 * j + 16 + t for j in range(h // 32) for t in range(16)],
        dtype=jnp.int32)

    bm1r = bm1.reshape(1, h)
    ba1r = ba1.reshape(1, h)
    bm2r = bm2.reshape(1, o)
    ba2r = ba2.reshape(1, o)

    xm1, xb1 = _stage_a_node(x, Wm1[:d], Wa1[h:], ba1r, bn)
    em1, em2 = _stage_a_edge(edge_attr, Wm1[d:][:, cperm], bm1r[:, cperm],
                             Wm2[h:][:, cperm], bm2r[:, cperm], be)

    agg1 = _sc_edge_pass(xm1, em1, src, dst, block_e=block_e)
    xm2, hb2 = _stage_b(agg1, xb1, Wa1[:h], Wm2[:h], Wa2[o:], ba2r, bn)

    agg2 = _sc_edge_pass(xm2, em2, src, dst, block_e=block_e)
    out = _stage_c(agg2, hb2, Wa2[:o], bn)
    return out


# speedup vs baseline: 1.1794x; 1.1794x over previous
import functools

import jax
import jax.numpy as jnp
from jax import lax
from jax.experimental import pallas as pl
from jax.experimental.pallas import tpu as pltpu
from jax.experimental.pallas import tpu_sc as plsc

F32 = jnp.float32
BF16 = jnp.bfloat16


def _dot(a, b):
    return jnp.dot(a, b, preferred_element_type=F32)


def _stage_a_node_body(x_ref, wm_ref, wab_ref, ba_ref, xm_ref, xb_ref):
    x = x_ref[...]
    xm_ref[...] = _dot(x, wm_ref[...])
    xb_ref[...] = _dot(x, wab_ref[...]) + ba_ref[...]


def _pack_bf16_pair(lo_f32, hi_f32):
    lo = lax.bitcast_convert_type(lo_f32.astype(BF16), jnp.uint16)
    hi = lax.bitcast_convert_type(hi_f32.astype(BF16), jnp.uint16)
    word = lo.astype(jnp.uint32) | (hi.astype(jnp.uint32) << 16)
    return lax.bitcast_convert_type(word, jnp.int32)


def _stage_a_edge_body(ea_ref, w1_ref, b1_ref, w2_ref, b2_ref,
                       em1_ref, em2_ref):
    ea = ea_ref[...]

    def em(w_ref, b_ref):
        w = w_ref[...]
        b = b_ref[...]
        half = w.shape[1] // 2
        lo = _dot(ea, w[:, :half]) + b[:, :half]
        hi = _dot(ea, w[:, half:]) + b[:, half:]
        return _pack_bf16_pair(lo, hi)

    em1_ref[...] = em(w1_ref, b1_ref)
    em2_ref[...] = em(w2_ref, b2_ref)


def _stage_b_body(agg_ref, xb_ref, wat_ref, wm2_ref, wa2b_ref, ba2_ref,
                  xm2_ref, hb2_ref):
    aggr = agg_ref[0] + agg_ref[1]
    h = jnp.maximum(_dot(aggr, wat_ref[...]) + xb_ref[...], 0.0)
    xm2_ref[...] = _dot(h, wm2_ref[...])
    hb2_ref[...] = _dot(h, wa2b_ref[...]) + ba2_ref[...]


def _stage_c_body(agg_ref, hb_ref, wat_ref, out_ref):
    aggr = agg_ref[0] + agg_ref[1]
    out_ref[...] = jnp.maximum(_dot(aggr, wat_ref[...]) + hb_ref[...], 0.0)


def _node_block(bn, h):
    return pl.BlockSpec((bn, h), lambda i: (i, 0))


def _full_block(shape):
    nd = len(shape)
    return pl.BlockSpec(shape, lambda i: (0,) * nd)


def _stage_a_node(x, wm_top, wa_bot, ba, bn):
    n, d = x.shape
    h = wm_top.shape[1]
    grid = n // bn
    return pl.pallas_call(
        _stage_a_node_body,
        grid=(grid,),
        in_specs=[_node_block(bn, d), _full_block(wm_top.shape),
                  _full_block(wa_bot.shape), _full_block(ba.shape)],
        out_specs=[_node_block(bn, h), _node_block(bn, h)],
        out_shape=[jax.ShapeDtypeStruct((n, h), F32),
                   jax.ShapeDtypeStruct((n, h), F32)],
    )(x, wm_top, wa_bot, ba)


def _stage_a_edge(ea, w1, b1, w2, b2, be):
    e, ed = ea.shape
    h = w1.shape[1]
    grid = e // be
    return pl.pallas_call(
        _stage_a_edge_body,
        grid=(grid,),
        in_specs=[_node_block(be, ed), _full_block(w1.shape),
                  _full_block(b1.shape), _full_block(w2.shape),
                  _full_block(b2.shape)],
        out_specs=[_node_block(be, h // 2), _node_block(be, h // 2)],
        out_shape=[jax.ShapeDtypeStruct((e, h // 2), jnp.int32),
                   jax.ShapeDtypeStruct((e, h // 2), jnp.int32)],
    )(ea, w1, b1, w2, b2)


def _stage_b(agg, xb, wa_top, wm2_top, wa2_bot, ba2, bn):
    n, h = xb.shape
    o = wm2_top.shape[1]
    grid = n // bn
    agg_spec = pl.BlockSpec((2, bn, h), lambda i: (0, i, 0))
    return pl.pallas_call(
        _stage_b_body,
        grid=(grid,),
        in_specs=[agg_spec, _node_block(bn, h), _full_block(wa_top.shape),
                  _full_block(wm2_top.shape), _full_block(wa2_bot.shape),
                  _full_block(ba2.shape)],
        out_specs=[_node_block(bn, o), _node_block(bn, o)],
        out_shape=[jax.ShapeDtypeStruct((n, o), F32),
                   jax.ShapeDtypeStruct((n, o), F32)],
    )(agg, xb, wa_top, wm2_top, wa2_bot, ba2)


def _stage_c(agg, hb, wa_top, bn):
    n, o = hb.shape
    grid = n // bn
    agg_spec = pl.BlockSpec((2, bn, o), lambda i: (0, i, 0))
    return pl.pallas_call(
        _stage_c_body,
        grid=(grid,),
        in_specs=[agg_spec, _node_block(bn, o), _full_block(wa_top.shape)],
        out_specs=_node_block(bn, o),
        out_shape=jax.ShapeDtypeStruct((n, o), F32),
    )(agg, hb, wa_top)


def _sc_edge_pass(xm, em, src, dst, *, block_e):
    n, h = xm.shape
    e = src.shape[0]
    mesh = plsc.VectorSubcoreMesh(core_axis_name="c", subcore_axis_name="s")
    n_tiles = mesh.num_cores * mesh.num_subcores
    n_sub = mesh.num_subcores
    chunk = e // n_tiles
    nblk = chunk // block_e
    assert chunk * n_tiles == e and nblk * block_e == chunk
    rows_base = (n // n_sub) // 8 * 8
    tail_rows = n - rows_base * n_sub
    assert rows_base % 8 == 0 and tail_rows % 8 == 0
    nfeat = h // 16

    @functools.partial(
        pl.kernel,
        out_type=jax.ShapeDtypeStruct((mesh.num_cores, n, h), F32),
        mesh=mesh,
        scratch_types=[
            pltpu.VMEM_SHARED((n, h), F32),
            pltpu.VMEM((chunk,), jnp.int32),
            pltpu.VMEM((chunk,), jnp.int32),
            pltpu.VMEM((block_e, h // 2), jnp.int32),
            pltpu.VMEM((block_e, h // 2), jnp.int32),
            pltpu.VMEM((block_e, h), F32),
            pltpu.VMEM((block_e, h), F32),
            pltpu.VMEM((block_e, h), F32),
            pltpu.SemaphoreType.DMA,
            pltpu.SemaphoreType.DMA,
            pltpu.SemaphoreType.DMA,
            pltpu.SemaphoreType.DMA,
            pltpu.SemaphoreType.DMA,
        ],
    )
    def edge_kernel(xm_hbm, em_hbm, src_hbm, dst_hbm, out_hbm,
                    acc, sidx, didx, em0, em1, gb0, gb1, obuf,
                    se0, se1, sg0, sg1, sem_s):
        embuf = (em0, em1)
        gbuf = (gb0, gb1)
        sem_e = (se0, se1)
        sem_g = (sg0, sg1)
        c = lax.axis_index("c")
        s = lax.axis_index("s")
        wid = c * n_sub + s

        zero = jnp.zeros((16,), F32)

        def zero_row(k, carry):
            for j in range(nfeat):
                obuf[k, pl.ds(j * 16, 16)] = zero
            return carry

        lax.fori_loop(0, block_e, zero_row, 0)
        r0 = s * rows_base
        nfull = rows_base // block_e
        rem = rows_base - nfull * block_e
        for t in range(nfull):
            pltpu.sync_copy(obuf, acc.at[pl.ds(r0 + t * block_e, block_e)])
        if rem:
            pltpu.sync_copy(obuf.at[pl.ds(0, rem)],
                            acc.at[pl.ds(r0 + nfull * block_e, rem)])
        if tail_rows:
            @pl.when(s == n_sub - 1)
            def _zero_tail():
                pltpu.sync_copy(obuf.at[pl.ds(0, tail_rows)],
                                acc.at[pl.ds(rows_base * n_sub, tail_rows)])

        base = wid * chunk
        pltpu.sync_copy(src_hbm.at[pl.ds(base, chunk)], sidx)
        pltpu.sync_copy(dst_hbm.at[pl.ds(base, chunk)], didx)
        plsc.subcore_barrier()

        def em_slice(i):
            return em_hbm.at[pl.ds(base + i * block_e, block_e)]

        def s_slice(i):
            return sidx.at[pl.ds(i * block_e, block_e)]

        def d_slice(i):
            return didx.at[pl.ds(i * block_e, block_e)]

        def compute(b):
            g, m = gbuf[b], embuf[b]

            def row(k, carry):
                for j in range(h // 32):
                    vw = m[k, pl.ds(j * 16, 16)]
                    ma = lax.bitcast_convert_type(
                        lax.shift_left(vw, 16), F32)
                    mb = lax.bitcast_convert_type(
                        lax.shift_left(
                            lax.shift_right_logical(vw, 16), 16), F32)
                    sla = pl.ds(j * 32, 16)
                    slb = pl.ds(j * 32 + 16, 16)
                    obuf[k, sla] = jnp.maximum(g[k, sla] + ma, 0.0)
                    obuf[k, slb] = jnp.maximum(g[k, slb] + mb, 0.0)
                return carry

            lax.fori_loop(0, block_e, row, 0)

        pltpu.async_copy(xm_hbm.at[s_slice(0)], gbuf[0], sem_g[0])
        pltpu.async_copy(em_slice(0), embuf[0], sem_e[0])

        def outer(t, carry):
            for b in range(2):
                i = t * 2 + b
                nxt = 1 - b

                @pl.when(i + 1 < nblk)
                def _issue_next():
                    pltpu.async_copy(xm_hbm.at[s_slice(i + 1)],
                                     gbuf[nxt], sem_g[nxt])
                    pltpu.async_copy(em_slice(i + 1),
                                     embuf[nxt], sem_e[nxt])

                pltpu.make_async_copy(xm_hbm.at[s_slice(i)],
                                      gbuf[b], sem_g[b]).wait()
                pltpu.make_async_copy(em_slice(i), embuf[b], sem_e[b]).wait()

                @pl.when(i > 0)
                def _wait_prev_scatter():
                    pltpu.make_async_copy(obuf, acc.at[d_slice(i - 1)],
                                          sem_s).wait()

                compute(b)
                pltpu.async_copy(obuf, acc.at[d_slice(i)], sem_s, add=True)
            return carry

        lax.fori_loop(0, (nblk + 1) // 2, outer, 0)
        last = nblk - 1
        pltpu.make_async_copy(obuf, acc.at[d_slice(last)], sem_s).wait()
        plsc.subcore_barrier()

        pltpu.sync_copy(acc.at[pl.ds(r0, rows_base)],
                        out_hbm.at[c, pl.ds(r0, rows_base)])
        if tail_rows:
            @pl.when(s == n_sub - 1)
            def _write_tail():
                pltpu.sync_copy(acc.at[pl.ds(rows_base * n_sub, tail_rows)],
                                out_hbm.at[c, pl.ds(rows_base * n_sub,
                                                    tail_rows)])

    return edge_kernel(xm, em, src, dst)


def kernel(x, edge_index, edge_attr, Wm1, bm1, Wa1, ba1, Wm2, bm2, Wa2, ba2):
    n, d = x.shape
    e, ed = edge_attr.shape
    h = Wm1.shape[1]
    o = Wm2.shape[1]
    src = edge_index[0]
    dst = edge_index[1]

    bn = 1000 if n % 1000 == 0 else 8
    be = 2000 if e % 2000 == 0 else 8
    block_e = 40

    cperm = jnp.asarray(
        [32 * j + t for j in range(h // 32) for t in range(16)]
        + [32 * j + 16 + t for j in range(h // 32) for t in range(16)],
        dtype=jnp.int32)

    bm1r = bm1.reshape(1, h)
    ba1r = ba1.reshape(1, h)
    bm2r = bm2.reshape(1, o)
    ba2r = ba2.reshape(1, o)

    xm1, xb1 = _stage_a_node(x, Wm1[:d], Wa1[h:], ba1r, bn)
    em1, em2 = _stage_a_edge(edge_attr, Wm1[d:][:, cperm], bm1r[:, cperm],
                             Wm2[h:][:, cperm], bm2r[:, cperm], be)

    agg1 = _sc_edge_pass(xm1, em1, src, dst, block_e=block_e)
    xm2, hb2 = _stage_b(agg1, xb1, Wa1[:h], Wm2[:h], Wa2[o:], ba2r, bn)

    agg2 = _sc_edge_pass(xm2, em2, src, dst, block_e=block_e)
    out = _stage_c(agg2, hb2, Wa2[:o], bn)
    return out
